# Initial kernel scaffold; baseline (speedup 1.0000x reference)
import jax
import jax.numpy as jnp
from jax.experimental import pallas as pl


def kernel(x, edge_index, W1, b1, Ek1, v1, W2, b2, Ek2, v2, W3, b3, Ek3, v3, W4, b4, Ek4, v4):
    raise NotImplementedError("write your pallas kernel here")


# speedup vs baseline: 12.5294x; 12.5294x over previous
import functools

import jax
import jax.numpy as jnp
from jax import lax
from jax.experimental import pallas as pl
from jax.experimental.pallas import tpu as pltpu
from jax.experimental.pallas import tpu_sc as plsc

NN = 10000
NNP = 10240
EE = 160000
NC = 2
NS = 16
KB = 400
EPT = EE // NS
NBATCH = EPT // KB
ROWS_PT = NNP // NS
KD = 200
DEG_EPT = EE // (NC * NS)
NBATCH_D = DEG_EPT // KD

_BM = 400
_GRID = NN // _BM


def _deg_body(dst_hbm, ones_hbm, zeros_hbm, out_hbm, idx_d, ones_v, acc):
    c = lax.axis_index("c")
    s = lax.axis_index("s")
    r0 = s * ROWS_PT
    pltpu.sync_copy(zeros_hbm.at[pl.ds(r0, ROWS_PT)], acc.at[pl.ds(r0, ROWS_PT)])
    pltpu.sync_copy(ones_hbm, ones_v)
    plsc.subcore_barrier()

    def body(b, carry):
        base = c * (EE // NC) + s * DEG_EPT + b * KD
        pltpu.sync_copy(dst_hbm.at[pl.ds(base, KD)], idx_d)
        pltpu.sync_copy(ones_v, acc.at[idx_d], add=True)
        return carry

    lax.fori_loop(0, NBATCH_D, body, 0)
    plsc.subcore_barrier()
    pltpu.sync_copy(acc.at[pl.ds(r0, ROWS_PT)],
                    out_hbm.at[pl.ds(c * NNP + r0, ROWS_PT)])


@functools.cache
def _deg_call():
    mesh = plsc.VectorSubcoreMesh(core_axis_name="c", subcore_axis_name="s")
    return pl.kernel(
        _deg_body,
        out_type=jax.ShapeDtypeStruct((NC * NNP, 16), jnp.float32),
        mesh=mesh,
        compiler_params=pltpu.CompilerParams(use_tc_tiling_on_sc=False),
        scratch_types=[
            pltpu.VMEM((KD,), jnp.int32),
            pltpu.VMEM((KD, 16), jnp.float32),
            pltpu.VMEM_SHARED((NNP, 16), jnp.float32),
        ],
    )


def _prop_body(kb, src2_hbm, dst_hbm, z_hbm, zeros_hbm, out_hbm,
               idx_s, idx_d, rows, sem, acc):
    c = lax.axis_index("c")
    s = lax.axis_index("s")
    r0 = s * ROWS_PT
    pltpu.sync_copy(zeros_hbm.at[pl.ds(r0, ROWS_PT)], acc.at[pl.ds(r0, ROWS_PT)])
    plsc.subcore_barrier()

    def body(b, carry):
        base = s * EPT + b * kb
        pltpu.sync_copy(src2_hbm.at[pl.ds(c * EE + base, kb)], idx_s)
        pltpu.sync_copy(dst_hbm.at[pl.ds(base, kb)], idx_d)
        pltpu.async_copy(z_hbm.at[idx_s], rows, sem).wait()
        pltpu.sync_copy(rows, acc.at[idx_d], add=True)
        return carry

    lax.fori_loop(0, EPT // kb, body, 0)
    plsc.subcore_barrier()
    pltpu.sync_copy(acc.at[pl.ds(r0, ROWS_PT)],
                    out_hbm.at[pl.ds(c * NNP + r0, ROWS_PT)])


@functools.cache
def _prop_call(d2):
    kb = 200 if d2 >= 128 else KB
    mesh = plsc.VectorSubcoreMesh(core_axis_name="c", subcore_axis_name="s")
    return pl.kernel(
        functools.partial(_prop_body, kb),
        out_type=jax.ShapeDtypeStruct((NC * NNP, d2), jnp.float32),
        mesh=mesh,
        compiler_params=pltpu.CompilerParams(use_tc_tiling_on_sc=False),
        scratch_types=[
            pltpu.VMEM((kb,), jnp.int32),
            pltpu.VMEM((kb,), jnp.int32),
            pltpu.VMEM((kb, d2), jnp.float32),
            pltpu.SemaphoreType.DMA,
            pltpu.VMEM_SHARED((NNP, d2), jnp.float32),
        ],
    )


def _alpha(ek, v):
    logits = jnp.sum(ek * v, axis=1, keepdims=True)
    m = jnp.max(logits)
    e = jnp.exp(logits - m)
    return e / jnp.sum(e)


def _mm_body(nparts, dout, d2, *refs):
    parts = refs[:nparts]
    ws = refs[nparts:2 * nparts]
    dinv_r, ek_r, v_r = refs[2 * nparts:2 * nparts + 3]
    z0_r, z1s_r, s2_r = refs[2 * nparts + 3:]
    acc = jnp.dot(parts[0][...], ws[0][...], preferred_element_type=jnp.float32)
    for p, w in zip(parts[1:], ws[1:]):
        acc = acc + jnp.dot(p[...], w[...], preferred_element_type=jnp.float32)
    al = _alpha(ek_r[...], v_r[...])
    dv = dinv_r[...]
    z0_r[...] = acc[:, :dout]
    z1s = acc[:, dout:2 * dout] * (al[1:2, :] * dv)
    s2 = acc[:, 2 * dout:] * (al[2:3, :] * dv)
    z1s_r[0] = z1s[:, :d2]
    z1s_r[1] = z1s[:, d2:]
    s2_r[0] = s2[:, :d2]
    s2_r[1] = s2[:, d2:]


def _mm_call(part_dims, dout):
    d2 = dout // 2
    nparts = len(part_dims)
    in_specs = (
        [pl.BlockSpec((_BM, dp), lambda i: (i, 0)) for dp in part_dims]
        + [pl.BlockSpec((dp, 3 * dout), lambda i: (0, 0)) for dp in part_dims]
        + [pl.BlockSpec((_BM, 1), lambda i: (i, 0)),
           pl.BlockSpec((3, 16), lambda i: (0, 0)),
           pl.BlockSpec((1, 16), lambda i: (0, 0))]
    )
    out_specs = (
        pl.BlockSpec((_BM, dout), lambda i: (i, 0)),
        pl.BlockSpec((NC, _BM, d2), lambda i: (0, i, 0)),
        pl.BlockSpec((NC, _BM, d2), lambda i: (0, i, 0)),
    )
    out_shape = (
        jax.ShapeDtypeStruct((NN, dout), jnp.float32),
        jax.ShapeDtypeStruct((NC, NN, d2), jnp.float32),
        jax.ShapeDtypeStruct((NC, NN, d2), jnp.float32),
    )
    return pl.pallas_call(
        functools.partial(_mm_body, nparts, dout, d2),
        grid=(_GRID,),
        in_specs=in_specs,
        out_specs=out_specs,
        out_shape=out_shape,
    )


def _comb_body(z1s_r, p2_r, dinv_r, s1_r):
    dv = dinv_r[...]
    dv2 = dv * dv
    s1_r[0] = z1s_r[0] + p2_r[0] * dv2
    s1_r[1] = z1s_r[1] + p2_r[1] * dv2


def _comb_call(dout):
    d2 = dout // 2
    spec3 = pl.BlockSpec((NC, _BM, d2), lambda i: (0, i, 0))
    return pl.pallas_call(
        _comb_body,
        grid=(_GRID,),
        in_specs=[spec3, spec3, pl.BlockSpec((_BM, 1), lambda i: (i, 0))],
        out_specs=spec3,
        out_shape=jax.ShapeDtypeStruct((NC, NN, d2), jnp.float32),
    )


def _fin_body(d2, z0_r, p1_r, dinv_r, b_r, ek_r, v_r, h_r):
    al = _alpha(ek_r[...], v_r[...])
    dv = dinv_r[...]
    ph = jnp.concatenate([p1_r[0], p1_r[1]], axis=1)
    t = al[0:1, :] * z0_r[...] + dv * ph + b_r[...]
    h_r[...] = jnp.where(t >= 0, t, 0.01 * t)


def _fin_call(dout):
    d2 = dout // 2
    return pl.pallas_call(
        functools.partial(_fin_body, d2),
        grid=(_GRID,),
        in_specs=[pl.BlockSpec((_BM, dout), lambda i: (i, 0)),
                  pl.BlockSpec((NC, _BM, d2), lambda i: (0, i, 0)),
                  pl.BlockSpec((_BM, 1), lambda i: (i, 0)),
                  pl.BlockSpec((1, dout), lambda i: (0, 0)),
                  pl.BlockSpec((3, 16), lambda i: (0, 0)),
                  pl.BlockSpec((1, 16), lambda i: (0, 0))],
        out_specs=pl.BlockSpec((_BM, dout), lambda i: (i, 0)),
        out_shape=jax.ShapeDtypeStruct((NN, dout), jnp.float32),
    )


def _dinv_body(da_r, dinv_r):
    da = da_r[...]
    deg = da[:NN, 0:1] + da[NNP:NNP + NN, 0:1]
    dinv_r[...] = lax.rsqrt(jnp.maximum(deg, 1.0))


@functools.cache
def _dinv_call():
    return pl.pallas_call(
        _dinv_body,
        out_shape=jax.ShapeDtypeStruct((NN, 1), jnp.float32),
    )


def _layer(parts, wcat, b, ek, v, src2, dst, dinv, dout):
    d2 = dout // 2
    part_dims = tuple(p.shape[1] for p in parts)
    wparts = []
    off = 0
    for dp in part_dims:
        wparts.append(wcat[off:off + dp])
        off += dp
    v2 = v.reshape(1, -1)
    z0, z1s, s2 = _mm_call(part_dims, dout)(*parts, *wparts, dinv, ek, v2)
    zeros = jnp.zeros((NNP, d2), jnp.float32)
    p2 = _prop_call(d2)(src2, dst, s2.reshape(NC * NN, d2), zeros)
    s1 = _comb_call(dout)(z1s, p2.reshape(NC, NNP, d2), dinv)
    p1 = _prop_call(d2)(src2, dst, s1.reshape(NC * NN, d2), zeros)
    h = _fin_call(dout)(z0, p1.reshape(NC, NNP, d2), dinv, b.reshape(1, -1), ek, v2)
    return h


def kernel(x, edge_index, W1, b1, Ek1, v1, W2, b2, Ek2, v2,
           W3, b3, Ek3, v3, W4, b4, Ek4, v4):
    src = edge_index[0]
    dst = edge_index[1]
    src2 = jnp.concatenate([src, src + NN])

    ones = jnp.ones((KD, 16), jnp.float32)
    zeros16 = jnp.zeros((NNP, 16), jnp.float32)
    deg_acc = _deg_call()(dst, ones, zeros16)
    dinv = _dinv_call()(deg_acc)

    W4p = jnp.pad(W4, ((0, 0), (0, 0), (0, 24)))
    b4p = jnp.pad(b4, (0, 24))

    def wcat(W):
        return jnp.concatenate([W[0], W[1], W[2]], axis=1)

    parts = [x]
    h1 = _layer(parts, wcat(W1), b1, Ek1, v1, src2, dst, dinv, 256)
    parts = [h1, x]
    h2 = _layer(parts, wcat(W2), b2, Ek2, v2, src2, dst, dinv, 128)
    parts = [h2, h1, x]
    h3 = _layer(parts, wcat(W3), b3, Ek3, v3, src2, dst, dinv, 64)
    parts = [h3, h2, h1, x]
    h4 = _layer(parts, wcat(W4p), b4p, Ek4, v4, src2, dst, dinv, 64)
    return h4[:, :40]


# speedup vs baseline: 14.6666x; 1.1706x over previous
import functools

import jax
import jax.numpy as jnp
from jax import lax
from jax.experimental import pallas as pl
from jax.experimental.pallas import tpu as pltpu
from jax.experimental.pallas import tpu_sc as plsc

NN = 10000
NNP = 10240
EE = 160000
NC = 2
NS = 16
KB = 200
EPT = EE // NS
NB = EPT // KB
ROWS_PT = NNP // NS
DEG_EPT = EE // (NC * NS)
NB_D = DEG_EPT // KB

_BM = 400
_GRID = NN // _BM


def _deg_body(dst_hbm, ones_hbm, zeros_hbm, out_hbm, idxd, ones_v, sems, acc):
    c = lax.axis_index("c")
    s = lax.axis_index("s")
    r0 = s * ROWS_PT
    t = c * NS + s
    pltpu.sync_copy(zeros_hbm.at[pl.ds(r0, ROWS_PT)], acc.at[pl.ds(r0, ROWS_PT)])
    pltpu.sync_copy(dst_hbm.at[pl.ds(t * NB_D, NB_D)], idxd)
    pltpu.sync_copy(ones_hbm, ones_v)
    plsc.subcore_barrier()

    def fire(b, carry):
        pltpu.async_copy(ones_v, acc.at[idxd.at[b]], sems, add=True)
        return carry

    lax.fori_loop(0, NB_D, fire, 0)

    def drain(b, carry):
        pltpu.make_async_copy(ones_v, acc.at[idxd.at[0]], sems).wait()
        return carry

    lax.fori_loop(0, NB_D, drain, 0)
    plsc.subcore_barrier()
    pltpu.sync_copy(acc.at[pl.ds(r0, ROWS_PT)],
                    out_hbm.at[pl.ds(c * NNP + r0, ROWS_PT)])


@functools.cache
def _deg_call():
    mesh = plsc.VectorSubcoreMesh(core_axis_name="c", subcore_axis_name="s")
    return pl.kernel(
        _deg_body,
        out_type=jax.ShapeDtypeStruct((NC * NNP, 16), jnp.float32),
        mesh=mesh,
        compiler_params=pltpu.CompilerParams(use_tc_tiling_on_sc=False),
        scratch_types=[
            pltpu.VMEM((NB_D, KB), jnp.int32),
            pltpu.VMEM((KB, 16), jnp.float32),
            pltpu.SemaphoreType.DMA,
            pltpu.VMEM_SHARED((NNP, 16), jnp.float32),
        ],
    )


def _prop_body(ngroups, d2, src_hbm, dst_hbm, z_hbm, zeros_hbm, out_hbm,
               idxs, idxd, rows0, rows1, semg0, semg1, sems0, sems1, acc):
    c = lax.axis_index("c")
    s = lax.axis_index("s")
    r0 = s * ROWS_PT
    pltpu.sync_copy(dst_hbm.at[pl.ds(s * NB, NB)], idxd)
    for g in range(ngroups):
        j = g * NC + c
        pltpu.sync_copy(zeros_hbm.at[pl.ds(r0, ROWS_PT)],
                        acc.at[pl.ds(r0, ROWS_PT)])
        pltpu.sync_copy(src_hbm.at[pl.ds(j * (NS * NB) + s * NB, NB)], idxs)
        plsc.subcore_barrier()
        pltpu.async_copy(z_hbm.at[idxs.at[0]], rows0, semg0)

        def body(i, carry):
            b0 = 2 * i
            b1 = 2 * i + 1
            pltpu.make_async_copy(z_hbm.at[idxs.at[b0]], rows0, semg0).wait()

            @pl.when(i > 0)
            def _():
                pltpu.make_async_copy(rows1, acc.at[idxd.at[0]], sems1).wait()

            pltpu.async_copy(z_hbm.at[idxs.at[b1]], rows1, semg1)
            pltpu.async_copy(rows0, acc.at[idxd.at[b0]], sems0, add=True)
            pltpu.make_async_copy(z_hbm.at[idxs.at[b1]], rows1, semg1).wait()

            @pl.when(i < NB // 2 - 1)
            def _():
                pltpu.make_async_copy(rows0, acc.at[idxd.at[0]], sems0).wait()
                pltpu.async_copy(z_hbm.at[idxs.at[b0 + 2]], rows0, semg0)

            pltpu.async_copy(rows1, acc.at[idxd.at[b1]], sems1, add=True)
            return carry

        lax.fori_loop(0, NB // 2, body, 0)
        pltpu.make_async_copy(rows0, acc.at[idxd.at[0]], sems0).wait()
        pltpu.make_async_copy(rows1, acc.at[idxd.at[0]], sems1).wait()
        plsc.subcore_barrier()
        pltpu.sync_copy(acc.at[pl.ds(r0, ROWS_PT)],
                        out_hbm.at[pl.ds(j * NNP + r0, ROWS_PT)])


@functools.cache
def _prop_call(d2, ngroups):
    mesh = plsc.VectorSubcoreMesh(core_axis_name="c", subcore_axis_name="s")
    return pl.kernel(
        functools.partial(_prop_body, ngroups, d2),
        out_type=jax.ShapeDtypeStruct((ngroups * NC * NNP, d2), jnp.float32),
        mesh=mesh,
        compiler_params=pltpu.CompilerParams(use_tc_tiling_on_sc=False),
        scratch_types=[
            pltpu.VMEM((NB, KB), jnp.int32),
            pltpu.VMEM((NB, KB), jnp.int32),
            pltpu.VMEM((KB, d2), jnp.float32),
            pltpu.VMEM((KB, d2), jnp.float32),
            pltpu.SemaphoreType.DMA,
            pltpu.SemaphoreType.DMA,
            pltpu.SemaphoreType.DMA,
            pltpu.SemaphoreType.DMA,
            pltpu.VMEM_SHARED((NNP, d2), jnp.float32),
        ],
    )


def _alpha(ek, v):
    logits = jnp.sum(ek * v, axis=1, keepdims=True)
    m = jnp.max(logits)
    e = jnp.exp(logits - m)
    return e / jnp.sum(e)


def _mm_body(nparts, dout, d2, *refs):
    nch = dout // d2
    parts = refs[:nparts]
    ws = refs[nparts:2 * nparts]
    dinv_r, ek_r, v_r = refs[2 * nparts:2 * nparts + 3]
    z0_r, z1s_r, s2_r = refs[2 * nparts + 3:]
    acc = jnp.dot(parts[0][...], ws[0][...], preferred_element_type=jnp.float32)
    for p, w in zip(parts[1:], ws[1:]):
        acc = acc + jnp.dot(p[...], w[...], preferred_element_type=jnp.float32)
    al = _alpha(ek_r[...], v_r[...])
    dv = dinv_r[...]
    z0_r[...] = acc[:, :dout]
    z1s = acc[:, dout:2 * dout] * (al[1:2, :] * dv)
    s2 = acc[:, 2 * dout:] * (al[2:3, :] * dv)
    for j in range(nch):
        z1s_r[j] = z1s[:, j * d2:(j + 1) * d2]
        s2_r[j] = s2[:, j * d2:(j + 1) * d2]


def _mm_call(part_dims, dout, d2):
    nch = dout // d2
    nparts = len(part_dims)
    in_specs = (
        [pl.BlockSpec((_BM, dp), lambda i: (i, 0)) for dp in part_dims]
        + [pl.BlockSpec((dp, 3 * dout), lambda i: (0, 0)) for dp in part_dims]
        + [pl.BlockSpec((_BM, 1), lambda i: (i, 0)),
           pl.BlockSpec((3, 16), lambda i: (0, 0)),
           pl.BlockSpec((1, 16), lambda i: (0, 0))]
    )
    out_specs = (
        pl.BlockSpec((_BM, dout), lambda i: (i, 0)),
        pl.BlockSpec((nch, _BM, d2), lambda i: (0, i, 0)),
        pl.BlockSpec((nch, _BM, d2), lambda i: (0, i, 0)),
    )
    out_shape = (
        jax.ShapeDtypeStruct((NN, dout), jnp.float32),
        jax.ShapeDtypeStruct((nch, NN, d2), jnp.float32),
        jax.ShapeDtypeStruct((nch, NN, d2), jnp.float32),
    )
    return pl.pallas_call(
        functools.partial(_mm_body, nparts, dout, d2),
        grid=(_GRID,),
        in_specs=in_specs,
        out_specs=out_specs,
        out_shape=out_shape,
    )


def _comb_body(nch, z1s_r, p2_r, dinv_r, s1_r):
    dv = dinv_r[...]
    dv2 = dv * dv
    for j in range(nch):
        s1_r[j] = z1s_r[j] + p2_r[j] * dv2


def _comb_call(dout, d2):
    nch = dout // d2
    spec = pl.BlockSpec((nch, _BM, d2), lambda i: (0, i, 0))
    return pl.pallas_call(
        functools.partial(_comb_body, nch),
        grid=(_GRID,),
        in_specs=[spec, spec, pl.BlockSpec((_BM, 1), lambda i: (i, 0))],
        out_specs=spec,
        out_shape=jax.ShapeDtypeStruct((nch, NN, d2), jnp.float32),
    )


def _fin_body(nch, z0_r, p1_r, dinv_r, b_r, ek_r, v_r, h_r):
    al = _alpha(ek_r[...], v_r[...])
    dv = dinv_r[...]
    ph = jnp.concatenate([p1_r[j] for j in range(nch)], axis=1)
    t = al[0:1, :] * z0_r[...] + dv * ph + b_r[...]
    h_r[...] = jnp.where(t >= 0, t, 0.01 * t)


def _fin_call(dout, d2):
    nch = dout // d2
    return pl.pallas_call(
        functools.partial(_fin_body, nch),
        grid=(_GRID,),
        in_specs=[pl.BlockSpec((_BM, dout), lambda i: (i, 0)),
                  pl.BlockSpec((nch, _BM, d2), lambda i: (0, i, 0)),
                  pl.BlockSpec((_BM, 1), lambda i: (i, 0)),
                  pl.BlockSpec((1, dout), lambda i: (0, 0)),
                  pl.BlockSpec((3, 16), lambda i: (0, 0)),
                  pl.BlockSpec((1, 16), lambda i: (0, 0))],
        out_specs=pl.BlockSpec((_BM, dout), lambda i: (i, 0)),
        out_shape=jax.ShapeDtypeStruct((NN, dout), jnp.float32),
    )


def _dinv_body(da_r, dinv_r):
    da = da_r[...]
    deg = da[:NN, 0:1] + da[NNP:NNP + NN, 0:1]
    dinv_r[...] = lax.rsqrt(jnp.maximum(deg, 1.0))


@functools.cache
def _dinv_call():
    return pl.pallas_call(
        _dinv_body,
        out_shape=jax.ShapeDtypeStruct((NN, 1), jnp.float32),
    )


def _layer(parts, wcat, b, ek, v, srcs, dst2d, dinv, dout, d2):
    nch = dout // d2
    ngroups = nch // NC
    part_dims = tuple(p.shape[1] for p in parts)
    wparts = []
    off = 0
    for dp in part_dims:
        wparts.append(wcat[off:off + dp])
        off += dp
    v2 = v.reshape(1, -1)
    z0, z1s, s2 = _mm_call(part_dims, dout, d2)(*parts, *wparts, dinv, ek, v2)
    zeros = jnp.zeros((NNP, d2), jnp.float32)
    prop = _prop_call(d2, ngroups)
    src2d = srcs[nch]
    p2 = prop(src2d, dst2d, s2.reshape(nch * NN, d2), zeros)
    s1 = _comb_call(dout, d2)(z1s, p2.reshape(nch, NNP, d2), dinv)
    p1 = prop(src2d, dst2d, s1.reshape(nch * NN, d2), zeros)
    h = _fin_call(dout, d2)(z0, p1.reshape(nch, NNP, d2), dinv,
                            b.reshape(1, -1), ek, v2)
    return h


def kernel(x, edge_index, W1, b1, Ek1, v1, W2, b2, Ek2, v2,
           W3, b3, Ek3, v3, W4, b4, Ek4, v4):
    src = edge_index[0]
    dst = edge_index[1]
    srcs = {
        2: jnp.concatenate([src, src + NN]).reshape(2 * NS * NB, KB),
        4: jnp.concatenate([src + j * NN for j in range(4)]).reshape(
            4 * NS * NB, KB),
    }
    dst2d = dst.reshape(NS * NB, KB)

    ones = jnp.ones((KB, 16), jnp.float32)
    zeros16 = jnp.zeros((NNP, 16), jnp.float32)
    deg_acc = _deg_call()(dst2d, ones, zeros16)
    dinv = _dinv_call()(deg_acc)

    W4p = jnp.pad(W4, ((0, 0), (0, 0), (0, 24)))
    b4p = jnp.pad(b4, (0, 24))

    def wcat(W):
        return jnp.concatenate([W[0], W[1], W[2]], axis=1)

    h1 = _layer([x], wcat(W1), b1, Ek1, v1, srcs, dst2d, dinv, 256, 64)
    h2 = _layer([h1, x], wcat(W2), b2, Ek2, v2, srcs, dst2d, dinv, 128, 64)
    h3 = _layer([h2, h1, x], wcat(W3), b3, Ek3, v3, srcs, dst2d, dinv, 64, 32)
    h4 = _layer([h3, h2, h1, x], wcat(W4p), b4p, Ek4, v4, srcs, dst2d, dinv,
                64, 32)
    return h4[:, :40]


# speedup vs baseline: 15.7049x; 1.0708x over previous
import functools

import jax
import jax.numpy as jnp
from jax import lax
from jax.experimental import pallas as pl
from jax.experimental.pallas import tpu as pltpu
from jax.experimental.pallas import tpu_sc as plsc

NN = 10000
NNP = 10240
EE = 160000
NC = 2
NS = 16
KB = 200
EPT = EE // NS
NB = EPT // KB
ROWS_PT = NNP // NS
DEG_EPT = EE // (NC * NS)
NB_D = DEG_EPT // KB

_BM = 400
_GRID = NN // _BM


def _deg_body(dst_hbm, ones_hbm, zeros_hbm, out_hbm, idxd, ones_v, sems, acc):
    c = lax.axis_index("c")
    s = lax.axis_index("s")
    r0 = s * ROWS_PT
    t = c * NS + s
    pltpu.sync_copy(zeros_hbm.at[pl.ds(r0, ROWS_PT)], acc.at[pl.ds(r0, ROWS_PT)])
    pltpu.sync_copy(dst_hbm.at[pl.ds(t * NB_D, NB_D)], idxd)
    pltpu.sync_copy(ones_hbm, ones_v)
    plsc.subcore_barrier()

    def fire(b, carry):
        pltpu.async_copy(ones_v, acc.at[idxd.at[b]], sems, add=True)
        return carry

    lax.fori_loop(0, NB_D, fire, 0)

    def drain(b, carry):
        pltpu.make_async_copy(ones_v, acc.at[idxd.at[0]], sems).wait()
        return carry

    lax.fori_loop(0, NB_D, drain, 0)
    plsc.subcore_barrier()
    pltpu.sync_copy(acc.at[pl.ds(r0, ROWS_PT)],
                    out_hbm.at[pl.ds(c * NNP + r0, ROWS_PT)])


@functools.cache
def _deg_call():
    mesh = plsc.VectorSubcoreMesh(core_axis_name="c", subcore_axis_name="s")
    return pl.kernel(
        _deg_body,
        out_type=jax.ShapeDtypeStruct((NC * NNP, 16), jnp.float32),
        mesh=mesh,
        compiler_params=pltpu.CompilerParams(use_tc_tiling_on_sc=False),
        scratch_types=[
            pltpu.VMEM((NB_D, KB), jnp.int32),
            pltpu.VMEM((KB, 16), jnp.float32),
            pltpu.SemaphoreType.DMA,
            pltpu.VMEM_SHARED((NNP, 16), jnp.float32),
        ],
    )


def _prop_body(ngroups, d2, kb, src_hbm, dst_hbm, z_hbm, zeros_hbm, out_hbm,
               idxs, idxd, rows0, rows1, semg0, semg1, sems0, sems1, acc):
    nb = EPT // kb
    c = lax.axis_index("c")
    s = lax.axis_index("s")
    r0 = s * ROWS_PT
    pltpu.sync_copy(dst_hbm.at[pl.ds(s * nb, nb)], idxd)
    for g in range(ngroups):
        j = g * NC + c
        pltpu.sync_copy(zeros_hbm.at[pl.ds(r0, ROWS_PT)],
                        acc.at[pl.ds(r0, ROWS_PT)])
        pltpu.sync_copy(src_hbm.at[pl.ds(j * (NS * nb) + s * nb, nb)], idxs)
        plsc.subcore_barrier()
        pltpu.async_copy(z_hbm.at[idxs.at[0]], rows0, semg0)

        def body(i, carry):
            b0 = 2 * i
            b1 = 2 * i + 1
            pltpu.make_async_copy(z_hbm.at[idxs.at[b0]], rows0, semg0).wait()

            @pl.when(i > 0)
            def _():
                pltpu.make_async_copy(rows1, acc.at[idxd.at[0]], sems1).wait()

            pltpu.async_copy(z_hbm.at[idxs.at[b1]], rows1, semg1)
            pltpu.async_copy(rows0, acc.at[idxd.at[b0]], sems0, add=True)
            pltpu.make_async_copy(z_hbm.at[idxs.at[b1]], rows1, semg1).wait()

            @pl.when(i < nb // 2 - 1)
            def _():
                pltpu.make_async_copy(rows0, acc.at[idxd.at[0]], sems0).wait()
                pltpu.async_copy(z_hbm.at[idxs.at[b0 + 2]], rows0, semg0)

            pltpu.async_copy(rows1, acc.at[idxd.at[b1]], sems1, add=True)
            return carry

        lax.fori_loop(0, nb // 2, body, 0)
        pltpu.make_async_copy(rows0, acc.at[idxd.at[0]], sems0).wait()
        pltpu.make_async_copy(rows1, acc.at[idxd.at[0]], sems1).wait()
        plsc.subcore_barrier()
        pltpu.sync_copy(acc.at[pl.ds(r0, ROWS_PT)],
                        out_hbm.at[pl.ds(j * NNP + r0, ROWS_PT)])


@functools.cache
def _prop_call(d2, ngroups):
    kb = 1000 if d2 == 32 else 200
    nb = EPT // kb
    mesh = plsc.VectorSubcoreMesh(core_axis_name="c", subcore_axis_name="s")
    return pl.kernel(
        functools.partial(_prop_body, ngroups, d2, kb),
        out_type=jax.ShapeDtypeStruct((ngroups * NC * NNP, d2), jnp.float32),
        mesh=mesh,
        compiler_params=pltpu.CompilerParams(use_tc_tiling_on_sc=False),
        scratch_types=[
            pltpu.VMEM((nb, kb), jnp.int32),
            pltpu.VMEM((nb, kb), jnp.int32),
            pltpu.VMEM((kb, d2), jnp.float32),
            pltpu.VMEM((kb, d2), jnp.float32),
            pltpu.SemaphoreType.DMA,
            pltpu.SemaphoreType.DMA,
            pltpu.SemaphoreType.DMA,
            pltpu.SemaphoreType.DMA,
            pltpu.VMEM_SHARED((NNP, d2), jnp.float32),
        ],
    )


KB1 = 40
NB1 = EPT // KB1


def _prop128_body(src3, dst3, z3, zeros_hbm, out3,
                  idxs, idxd, rows0, rows1, semg0, semg1, sems0, sems1, acc):
    c = lax.axis_index("c")
    s = lax.axis_index("s")
    r0 = s * ROWS_PT
    pltpu.sync_copy(zeros_hbm.at[pl.ds(r0, ROWS_PT)], acc.at[pl.ds(r0, ROWS_PT)])
    pltpu.sync_copy(src3.at[s], idxs)
    pltpu.sync_copy(dst3.at[s], idxd)
    plsc.subcore_barrier()
    pltpu.async_copy(z3.at[c].at[idxs.at[0]], rows0, semg0)

    def body(i, carry):
        b0 = 2 * i
        b1 = 2 * i + 1
        pltpu.make_async_copy(z3.at[c].at[idxs.at[b0]], rows0, semg0).wait()

        @pl.when(i > 0)
        def _():
            pltpu.make_async_copy(rows1, acc.at[idxd.at[0]], sems1).wait()

        pltpu.async_copy(z3.at[c].at[idxs.at[b1]], rows1, semg1)
        pltpu.async_copy(rows0, acc.at[idxd.at[b0]], sems0, add=True)
        pltpu.make_async_copy(z3.at[c].at[idxs.at[b1]], rows1, semg1).wait()

        @pl.when(i < NB1 // 2 - 1)
        def _():
            pltpu.make_async_copy(rows0, acc.at[idxd.at[0]], sems0).wait()
            pltpu.async_copy(z3.at[c].at[idxs.at[b0 + 2]], rows0, semg0)

        pltpu.async_copy(rows1, acc.at[idxd.at[b1]], sems1, add=True)
        return carry

    lax.fori_loop(0, NB1 // 2, body, 0)
    pltpu.make_async_copy(rows0, acc.at[idxd.at[0]], sems0).wait()
    pltpu.make_async_copy(rows1, acc.at[idxd.at[0]], sems1).wait()
    plsc.subcore_barrier()
    pltpu.sync_copy(acc.at[pl.ds(r0, ROWS_PT)],
                    out3.at[c].at[pl.ds(r0, ROWS_PT)])


@functools.cache
def _prop128_call():
    mesh = plsc.VectorSubcoreMesh(core_axis_name="c", subcore_axis_name="s")
    return pl.kernel(
        _prop128_body,
        out_type=jax.ShapeDtypeStruct((NC, NNP, 128), jnp.float32),
        mesh=mesh,
        scratch_types=[
            pltpu.VMEM((NB1, KB1), jnp.int32),
            pltpu.VMEM((NB1, KB1), jnp.int32),
            pltpu.VMEM((KB1, 128), jnp.float32),
            pltpu.VMEM((KB1, 128), jnp.float32),
            pltpu.SemaphoreType.DMA,
            pltpu.SemaphoreType.DMA,
            pltpu.SemaphoreType.DMA,
            pltpu.SemaphoreType.DMA,
            pltpu.VMEM_SHARED((NNP, 128), jnp.float32),
        ],
    )


def _alpha(ek, v):
    logits = jnp.sum(ek * v, axis=1, keepdims=True)
    m = jnp.max(logits)
    e = jnp.exp(logits - m)
    return e / jnp.sum(e)


def _mm_body(nparts, dout, d2, *refs):
    nch = dout // d2
    parts = refs[:nparts]
    ws = refs[nparts:2 * nparts]
    dinv_r, ek_r, v_r = refs[2 * nparts:2 * nparts + 3]
    z0_r, z1s_r, s2_r = refs[2 * nparts + 3:]
    acc = jnp.dot(parts[0][...], ws[0][...], preferred_element_type=jnp.float32)
    for p, w in zip(parts[1:], ws[1:]):
        acc = acc + jnp.dot(p[...], w[...], preferred_element_type=jnp.float32)
    al = _alpha(ek_r[...], v_r[...])
    dv = dinv_r[...]
    z0_r[...] = acc[:, :dout]
    z1s = acc[:, dout:2 * dout] * (al[1:2, :] * dv)
    s2 = acc[:, 2 * dout:] * (al[2:3, :] * dv)
    for j in range(nch):
        z1s_r[j] = z1s[:, j * d2:(j + 1) * d2]
        s2_r[j] = s2[:, j * d2:(j + 1) * d2]


def _mm_call(part_dims, dout, d2):
    nch = dout // d2
    nparts = len(part_dims)
    in_specs = (
        [pl.BlockSpec((_BM, dp), lambda i: (i, 0)) for dp in part_dims]
        + [pl.BlockSpec((dp, 3 * dout), lambda i: (0, 0)) for dp in part_dims]
        + [pl.BlockSpec((_BM, 1), lambda i: (i, 0)),
           pl.BlockSpec((3, 16), lambda i: (0, 0)),
           pl.BlockSpec((1, 16), lambda i: (0, 0))]
    )
    out_specs = (
        pl.BlockSpec((_BM, dout), lambda i: (i, 0)),
        pl.BlockSpec((nch, _BM, d2), lambda i: (0, i, 0)),
        pl.BlockSpec((nch, _BM, d2), lambda i: (0, i, 0)),
    )
    out_shape = (
        jax.ShapeDtypeStruct((NN, dout), jnp.float32),
        jax.ShapeDtypeStruct((nch, NN, d2), jnp.float32),
        jax.ShapeDtypeStruct((nch, NN, d2), jnp.float32),
    )
    return pl.pallas_call(
        functools.partial(_mm_body, nparts, dout, d2),
        grid=(_GRID,),
        in_specs=in_specs,
        out_specs=out_specs,
        out_shape=out_shape,
    )


def _comb_body(nch, z1s_r, p2_r, dinv_r, s1_r):
    dv = dinv_r[...]
    dv2 = dv * dv
    for j in range(nch):
        s1_r[j] = z1s_r[j] + p2_r[j] * dv2


def _comb_call(dout, d2):
    nch = dout // d2
    spec = pl.BlockSpec((nch, _BM, d2), lambda i: (0, i, 0))
    return pl.pallas_call(
        functools.partial(_comb_body, nch),
        grid=(_GRID,),
        in_specs=[spec, spec, pl.BlockSpec((_BM, 1), lambda i: (i, 0))],
        out_specs=spec,
        out_shape=jax.ShapeDtypeStruct((nch, NN, d2), jnp.float32),
    )


def _fin_body(nch, z0_r, p1_r, dinv_r, b_r, ek_r, v_r, h_r):
    al = _alpha(ek_r[...], v_r[...])
    dv = dinv_r[...]
    ph = jnp.concatenate([p1_r[j] for j in range(nch)], axis=1)
    t = al[0:1, :] * z0_r[...] + dv * ph + b_r[...]
    h_r[...] = jnp.where(t >= 0, t, 0.01 * t)


def _fin_call(dout, d2):
    nch = dout // d2
    return pl.pallas_call(
        functools.partial(_fin_body, nch),
        grid=(_GRID,),
        in_specs=[pl.BlockSpec((_BM, dout), lambda i: (i, 0)),
                  pl.BlockSpec((nch, _BM, d2), lambda i: (0, i, 0)),
                  pl.BlockSpec((_BM, 1), lambda i: (i, 0)),
                  pl.BlockSpec((1, dout), lambda i: (0, 0)),
                  pl.BlockSpec((3, 16), lambda i: (0, 0)),
                  pl.BlockSpec((1, 16), lambda i: (0, 0))],
        out_specs=pl.BlockSpec((_BM, dout), lambda i: (i, 0)),
        out_shape=jax.ShapeDtypeStruct((NN, dout), jnp.float32),
    )


def _dinv_body(da_r, dinv_r):
    da = da_r[...]
    deg = da[:NN, 0:1] + da[NNP:NNP + NN, 0:1]
    dinv_r[...] = lax.rsqrt(jnp.maximum(deg, 1.0))


@functools.cache
def _dinv_call():
    return pl.pallas_call(
        _dinv_body,
        out_shape=jax.ShapeDtypeStruct((NN, 1), jnp.float32),
    )


def _layer(parts, wcat, b, ek, v, srcs, dst2d, dinv, dout, d2):
    nch = dout // d2
    ngroups = nch // NC
    part_dims = tuple(p.shape[1] for p in parts)
    wparts = []
    off = 0
    for dp in part_dims:
        wparts.append(wcat[off:off + dp])
        off += dp
    v2 = v.reshape(1, -1)
    z0, z1s, s2 = _mm_call(part_dims, dout, d2)(*parts, *wparts, dinv, ek, v2)
    zeros = jnp.zeros((NNP, d2), jnp.float32)
    if d2 == 128:
        src3, dst3 = srcs['tc128']
        prop = _prop128_call()
        p2 = prop(src3, dst3, s2, zeros)
        s1 = _comb_call(dout, d2)(z1s, p2, dinv)
        p1 = prop(src3, dst3, s1, zeros)
    else:
        kb = 1000 if d2 == 32 else 200
        prop = _prop_call(d2, ngroups)
        src2d, dst2dk = srcs[(nch, kb)]
        p2 = prop(src2d, dst2dk, s2.reshape(nch * NN, d2), zeros)
        s1 = _comb_call(dout, d2)(z1s, p2.reshape(nch, NNP, d2), dinv)
        p1 = prop(src2d, dst2dk, s1.reshape(nch * NN, d2), zeros)
        p1 = p1.reshape(nch, NNP, d2)
    h = _fin_call(dout, d2)(z0, p1.reshape(nch, NNP, d2), dinv,
                            b.reshape(1, -1), ek, v2)
    return h


def kernel(x, edge_index, W1, b1, Ek1, v1, W2, b2, Ek2, v2,
           W3, b3, Ek3, v3, W4, b4, Ek4, v4):
    src = edge_index[0]
    dst = edge_index[1]
    src2 = jnp.concatenate([src, src + NN])
    srcs = {
        'tc128': (src.reshape(NS, NB1, KB1), dst.reshape(NS, NB1, KB1)),
        (2, 200): (src2.reshape(2 * NS * 50, 200), dst.reshape(NS * 50, 200)),
        (2, 1000): (src2.reshape(2 * NS * 10, 1000),
                    dst.reshape(NS * 10, 1000)),
        (4, 200): (jnp.concatenate([src + j * NN for j in range(4)]).reshape(
            4 * NS * 50, 200), dst.reshape(NS * 50, 200)),
    }
    dst2d = dst.reshape(NS * NB, KB)

    ones = jnp.ones((KB, 16), jnp.float32)
    zeros16 = jnp.zeros((NNP, 16), jnp.float32)
    deg_acc = _deg_call()(dst2d, ones, zeros16)
    dinv = _dinv_call()(deg_acc)

    W4p = jnp.pad(W4, ((0, 0), (0, 0), (0, 24)))
    b4p = jnp.pad(b4, (0, 24))

    def wcat(W):
        return jnp.concatenate([W[0], W[1], W[2]], axis=1)

    h1 = _layer([x], wcat(W1), b1, Ek1, v1, srcs, dst2d, dinv, 256, 64)
    h2 = _layer([h1, x], wcat(W2), b2, Ek2, v2, srcs, dst2d, dinv, 128, 64)
    h3 = _layer([h2, h1, x], wcat(W3), b3, Ek3, v3, srcs, dst2d, dinv, 64, 32)
    h4 = _layer([h3, h2, h1, x], wcat(W4p), b4p, Ek4, v4, srcs, dst2d, dinv,
                64, 32)
    return h4[:, :40]


# speedup vs baseline: 17.5407x; 1.1169x over previous
import functools

import jax
import jax.numpy as jnp
from jax import lax
from jax.experimental import pallas as pl
from jax.experimental.pallas import tpu as pltpu
from jax.experimental.pallas import tpu_sc as plsc

NN = 10000
NNP = 10240
EE = 160000
NC = 2
NS = 16
EPT = EE // NS
ROWS_PT = NNP // NS
KB_D = 200
DEG_EPT = EE // (NC * NS)
NB_D = DEG_EPT // KB_D
FC = 160

_BM = 400
_GRID = NN // _BM


def _deg_body(dst_hbm, ones_hbm, zeros_hbm, out_hbm, idxd, ones_v, sems, acc):
    c = lax.axis_index("c")
    s = lax.axis_index("s")
    r0 = s * ROWS_PT
    t = c * NS + s
    pltpu.sync_copy(zeros_hbm.at[pl.ds(r0, ROWS_PT)], acc.at[pl.ds(r0, ROWS_PT)])
    pltpu.sync_copy(dst_hbm.at[pl.ds(t * NB_D, NB_D)], idxd)
    pltpu.sync_copy(ones_hbm, ones_v)
    plsc.subcore_barrier()

    def fire(b, carry):
        pltpu.async_copy(ones_v, acc.at[idxd.at[b]], sems, add=True)
        return carry

    lax.fori_loop(0, NB_D, fire, 0)

    def drain(b, carry):
        pltpu.make_async_copy(ones_v, acc.at[idxd.at[0]], sems).wait()
        return carry

    lax.fori_loop(0, NB_D, drain, 0)
    plsc.subcore_barrier()
    pltpu.sync_copy(acc.at[pl.ds(r0, ROWS_PT)],
                    out_hbm.at[pl.ds(c * NNP + r0, ROWS_PT)])


@functools.cache
def _deg_call():
    mesh = plsc.VectorSubcoreMesh(core_axis_name="c", subcore_axis_name="s")
    return pl.kernel(
        _deg_body,
        out_type=jax.ShapeDtypeStruct((NC * NNP, 16), jnp.float32),
        mesh=mesh,
        compiler_params=pltpu.CompilerParams(use_tc_tiling_on_sc=False),
        scratch_types=[
            pltpu.VMEM((NB_D, KB_D), jnp.int32),
            pltpu.VMEM((KB_D, 16), jnp.float32),
            pltpu.SemaphoreType.DMA,
            pltpu.VMEM_SHARED((NNP, 16), jnp.float32),
        ],
    )


def _prop_f_body(ngroups, d2, kb, leaky,
                 src_hbm, dst_hbm, z_hbm, init_hbm, scale_hbm, out_hbm,
                 idxs, idxd, rows0, rows1, scale_v,
                 semg0, semg1, sems0, sems1, acc):
    nb = EPT // kb
    c = lax.axis_index("c")
    s = lax.axis_index("s")
    r0 = s * ROWS_PT
    pltpu.sync_copy(dst_hbm.at[pl.ds(s * nb, nb)], idxd)
    pltpu.sync_copy(scale_hbm.at[pl.ds(r0, ROWS_PT)], scale_v)
    for g in range(ngroups):
        j = g * NC + c
        pltpu.sync_copy(init_hbm.at[pl.ds(j * NNP + r0, ROWS_PT)],
                        acc.at[pl.ds(r0, ROWS_PT)])
        pltpu.sync_copy(src_hbm.at[pl.ds(j * (NS * nb) + s * nb, nb)], idxs)
        plsc.subcore_barrier()
        pltpu.async_copy(z_hbm.at[idxs.at[0]], rows0, semg0)

        def body(i, carry):
            b0 = 2 * i
            b1 = 2 * i + 1
            pltpu.make_async_copy(z_hbm.at[idxs.at[b0]], rows0, semg0).wait()

            @pl.when(i > 0)
            def _():
                pltpu.make_async_copy(rows1, acc.at[idxd.at[0]], sems1).wait()

            pltpu.async_copy(z_hbm.at[idxs.at[b1]], rows1, semg1)
            pltpu.async_copy(rows0, acc.at[idxd.at[b0]], sems0, add=True)
            pltpu.make_async_copy(z_hbm.at[idxs.at[b1]], rows1, semg1).wait()

            @pl.when(i < nb // 2 - 1)
            def _():
                pltpu.make_async_copy(rows0, acc.at[idxd.at[0]], sems0).wait()
                pltpu.async_copy(z_hbm.at[idxs.at[b0 + 2]], rows0, semg0)

            pltpu.async_copy(rows1, acc.at[idxd.at[b1]], sems1, add=True)
            return carry

        lax.fori_loop(0, nb // 2, body, 0)
        pltpu.make_async_copy(rows0, acc.at[idxd.at[0]], sems0).wait()
        pltpu.make_async_copy(rows1, acc.at[idxd.at[0]], sems1).wait()
        plsc.subcore_barrier()
        for m in range(ROWS_PT // FC):
            pltpu.sync_copy(acc.at[pl.ds(r0 + m * FC, FC)],
                            rows0.at[pl.ds(0, FC)])

            def srow(r, carry):
                idxv = jnp.full((16,), m * FC + r, jnp.int32)
                sc = plsc.load_gather(scale_v, [idxv])
                for jj in range(d2 // 16):
                    vec = rows0[r, pl.ds(jj * 16, 16)] * sc
                    if leaky:
                        vec = jnp.where(vec >= 0, vec, 0.01 * vec)
                    rows0[r, pl.ds(jj * 16, 16)] = vec
                return carry

            lax.fori_loop(0, FC, srow, 0)
            pltpu.sync_copy(rows0.at[pl.ds(0, FC)],
                            out_hbm.at[pl.ds(j * NNP + r0 + m * FC, FC)])


@functools.cache
def _prop_f_call(d2, ngroups, leaky):
    kb = 1000 if d2 == 32 else 200
    nb = EPT // kb
    mesh = plsc.VectorSubcoreMesh(core_axis_name="c", subcore_axis_name="s")
    return pl.kernel(
        functools.partial(_prop_f_body, ngroups, d2, kb, leaky),
        out_type=jax.ShapeDtypeStruct((ngroups * NC * NNP, d2), jnp.float32),
        mesh=mesh,
        compiler_params=pltpu.CompilerParams(use_tc_tiling_on_sc=False,
                                             needs_layout_passes=False),
        scratch_types=[
            pltpu.VMEM((nb, kb), jnp.int32),
            pltpu.VMEM((nb, kb), jnp.int32),
            pltpu.VMEM((kb, d2), jnp.float32),
            pltpu.VMEM((kb, d2), jnp.float32),
            pltpu.VMEM((ROWS_PT,), jnp.float32),
            pltpu.SemaphoreType.DMA,
            pltpu.SemaphoreType.DMA,
            pltpu.SemaphoreType.DMA,
            pltpu.SemaphoreType.DMA,
            pltpu.VMEM_SHARED((NNP, d2), jnp.float32),
        ],
    )


def _alpha(ek, v):
    logits = jnp.sum(ek * v, axis=1, keepdims=True)
    m = jnp.max(logits)
    e = jnp.exp(logits - m)
    return e / jnp.sum(e)


def _mm_body(pspec, dout, d2, *refs):
    nch = dout // d2
    nparts = len(pspec)
    parts = refs[:nparts]
    ws = refs[nparts:2 * nparts]
    dinv_r, ek_r, v_r, b_r = refs[2 * nparts:2 * nparts + 4]
    z0m_r, z1m_r, s2_r = refs[2 * nparts + 4:]
    acc = None
    for p, w, kind in zip(parts, ws, pspec):
        pv = p[...]
        if kind[0] == '3d':
            pv = pv[0]
        d = jnp.dot(pv, w[...], preferred_element_type=jnp.float32)
        acc = d if acc is None else acc + d
    al = _alpha(ek_r[...], v_r[...])
    dv = dinv_r[...]
    idv = 1.0 / dv
    z0m = acc[:, :dout] * (al[0:1, :] * idv) + idv * b_r[...]
    z1m = acc[:, dout:2 * dout] * (al[1:2, :] * idv)
    s2 = acc[:, 2 * dout:] * (al[2:3, :] * dv)
    for j in range(nch):
        z0m_r[j] = z0m[:, j * d2:(j + 1) * d2]
        z1m_r[j] = z1m[:, j * d2:(j + 1) * d2]
        s2_r[j] = s2[:, j * d2:(j + 1) * d2]


@functools.cache
def _mm_call(pspec, dout, d2):
    nch = dout // d2
    in_specs = []
    for kind in pspec:
        if kind[0] == '2d':
            in_specs.append(pl.BlockSpec((_BM, kind[1]), lambda i: (i, 0)))
        else:
            jj = kind[2]
            in_specs.append(pl.BlockSpec((1, _BM, kind[1]),
                                         lambda i, jj=jj: (jj, i, 0)))
    for kind in pspec:
        in_specs.append(pl.BlockSpec((kind[1], 3 * dout), lambda i: (0, 0)))
    in_specs += [pl.BlockSpec((_BM, 1), lambda i: (i, 0)),
                 pl.BlockSpec((3, 16), lambda i: (0, 0)),
                 pl.BlockSpec((1, 16), lambda i: (0, 0)),
                 pl.BlockSpec((1, dout), lambda i: (0, 0))]
    spec_np = pl.BlockSpec((nch, _BM, d2), lambda i: (0, i, 0))
    out_specs = (spec_np, spec_np, spec_np)
    out_shape = (
        jax.ShapeDtypeStruct((nch, NNP, d2), jnp.float32),
        jax.ShapeDtypeStruct((nch, NNP, d2), jnp.float32),
        jax.ShapeDtypeStruct((nch, NN, d2), jnp.float32),
    )
    return pl.pallas_call(
        functools.partial(_mm_body, pspec, dout, d2),
        grid=(_GRID,),
        in_specs=in_specs,
        out_specs=out_specs,
        out_shape=out_shape,
    )


def _dinv_body(da_r, dinv_r, dvsq_r):
    da = da_r[...]
    deg = da[:NNP, 0:1] + da[NNP:, 0:1]
    dv = lax.rsqrt(jnp.maximum(deg, 1.0))
    dinv_r[...] = dv
    dvsq_r[...] = dv * dv


@functools.cache
def _dinv_call():
    return pl.pallas_call(
        _dinv_body,
        out_shape=(jax.ShapeDtypeStruct((NNP, 1), jnp.float32),
                   jax.ShapeDtypeStruct((NNP, 1), jnp.float32)),
    )


def _layer(parts, wcat, b, ek, v, srcs, dinv, dv1, dv2, dout, d2):
    nch = dout // d2
    ngroups = nch // NC
    kb = 1000 if d2 == 32 else 200
    pspec = []
    ops = []
    wparts = []
    off = 0
    for f in parts:
        if f.ndim == 2:
            dp = f.shape[1]
            pspec.append(('2d', dp))
            ops.append(f)
            wparts.append(wcat[off:off + dp])
            off += dp
        else:
            nchp, _, d2p = f.shape
            for j in range(nchp):
                pspec.append(('3d', d2p, j))
                ops.append(f)
                wparts.append(wcat[off:off + d2p])
                off += d2p
    v2 = v.reshape(1, -1)
    z0m, z1m, s2 = _mm_call(tuple(pspec), dout, d2)(
        *ops, *wparts, dinv, ek, v2, b.reshape(1, -1))
    srcN, dstk = srcs[(nch, kb, 'N')]
    srcP = srcs[(nch, kb, 'P')]
    s1c = _prop_f_call(d2, ngroups, False)(
        srcN, dstk, s2.reshape(nch * NN, d2), z1m.reshape(nch * NNP, d2), dv2)
    h = _prop_f_call(d2, ngroups, True)(
        srcP, dstk, s1c, z0m.reshape(nch * NNP, d2), dv1)
    return h.reshape(nch, NNP, d2)


def kernel(x, edge_index, W1, b1, Ek1, v1, W2, b2, Ek2, v2,
           W3, b3, Ek3, v3, W4, b4, Ek4, v4):
    src = edge_index[0]
    dst = edge_index[1]
    srcN2 = jnp.concatenate([src, src + NN])
    srcP2 = jnp.concatenate([src, src + NNP])
    srcs = {
        (4, 200, 'N'): (jnp.concatenate(
            [src + j * NN for j in range(4)]).reshape(4 * NS * 50, 200),
            dst.reshape(NS * 50, 200)),
        (4, 200, 'P'): jnp.concatenate(
            [src + j * NNP for j in range(4)]).reshape(4 * NS * 50, 200),
        (2, 200, 'N'): (srcN2.reshape(2 * NS * 50, 200),
                        dst.reshape(NS * 50, 200)),
        (2, 200, 'P'): srcP2.reshape(2 * NS * 50, 200),
        (2, 1000, 'N'): (srcN2.reshape(2 * NS * 10, 1000),
                         dst.reshape(NS * 10, 1000)),
        (2, 1000, 'P'): srcP2.reshape(2 * NS * 10, 1000),
    }

    ones = jnp.ones((KB_D, 16), jnp.float32)
    zeros16 = jnp.zeros((NNP, 16), jnp.float32)
    deg_acc = _deg_call()(dst.reshape(NS * 50, KB_D), ones, zeros16)
    dinv, dvsq = _dinv_call()(deg_acc)
    dv1 = dinv.reshape(NNP)
    dv2 = dvsq.reshape(NNP)

    W4p = jnp.pad(W4, ((0, 0), (0, 0), (0, 24)))
    b4p = jnp.pad(b4, (0, 24))

    def wcat(W):
        return jnp.concatenate([W[0], W[1], W[2]], axis=1)

    h1 = _layer([x], wcat(W1), b1, Ek1, v1, srcs, dinv, dv1, dv2, 256, 64)
    h2 = _layer([h1, x], wcat(W2), b2, Ek2, v2, srcs, dinv, dv1, dv2, 128, 64)
    h3 = _layer([h2, h1, x], wcat(W3), b3, Ek3, v3, srcs, dinv, dv1, dv2,
                64, 32)
    h4 = _layer([h3, h2, h1, x], wcat(W4p), b4p, Ek4, v4, srcs, dinv, dv1, dv2,
                64, 32)
    return jnp.concatenate([h4[0, :NN], h4[1, :NN]], axis=1)[:, :40]


# speedup vs baseline: 17.8161x; 1.0157x over previous
import functools

import jax
import jax.numpy as jnp
from jax import lax
from jax.experimental import pallas as pl
from jax.experimental.pallas import tpu as pltpu
from jax.experimental.pallas import tpu_sc as plsc

NN = 10000
NNP = 10240
EE = 160000
NC = 2
NS = 16
EPT = EE // NS
ROWS_PT = NNP // NS
KB_D = 200
DEG_EPT = EE // (NC * NS)
NB_D = DEG_EPT // KB_D
FC = 160

_BM = 400
_GRID = NN // _BM


def _deg_body(dst_hbm, ones_hbm, zeros_hbm, out_hbm, idxd, ones_v, sems, acc):
    c = lax.axis_index("c")
    s = lax.axis_index("s")
    r0 = s * ROWS_PT
    t = c * NS + s
    pltpu.sync_copy(zeros_hbm.at[pl.ds(r0, ROWS_PT)], acc.at[pl.ds(r0, ROWS_PT)])
    pltpu.sync_copy(dst_hbm.at[pl.ds(t * NB_D, NB_D)], idxd)
    pltpu.sync_copy(ones_hbm, ones_v)
    plsc.subcore_barrier()

    def fire(b, carry):
        pltpu.async_copy(ones_v, acc.at[idxd.at[b]], sems, add=True)
        return carry

    lax.fori_loop(0, NB_D, fire, 0)

    def drain(b, carry):
        pltpu.make_async_copy(ones_v, acc.at[idxd.at[0]], sems).wait()
        return carry

    lax.fori_loop(0, NB_D, drain, 0)
    plsc.subcore_barrier()
    pltpu.sync_copy(acc.at[pl.ds(r0, ROWS_PT)],
                    out_hbm.at[pl.ds(c * NNP + r0, ROWS_PT)])


@functools.cache
def _deg_call():
    mesh = plsc.VectorSubcoreMesh(core_axis_name="c", subcore_axis_name="s")
    return pl.kernel(
        _deg_body,
        out_type=jax.ShapeDtypeStruct((NC * NNP, 16), jnp.float32),
        mesh=mesh,
        compiler_params=pltpu.CompilerParams(use_tc_tiling_on_sc=False),
        scratch_types=[
            pltpu.VMEM((NB_D, KB_D), jnp.int32),
            pltpu.VMEM((KB_D, 16), jnp.float32),
            pltpu.SemaphoreType.DMA,
            pltpu.VMEM_SHARED((NNP, 16), jnp.float32),
        ],
    )


def _prop_f_body(ngroups, d2, kb, leaky,
                 src_hbm, dst_hbm, z_hbm, init_hbm, scale_hbm, out_hbm,
                 idxs, idxd, rows0, rows1, scale_v,
                 semg0, semg1, sems0, sems1, acc):
    nb = EPT // kb
    c = lax.axis_index("c")
    s = lax.axis_index("s")
    r0 = s * ROWS_PT
    pltpu.sync_copy(dst_hbm.at[pl.ds(s * nb, nb)], idxd)
    pltpu.sync_copy(src_hbm.at[pl.ds(s * nb, nb)], idxs)
    pltpu.sync_copy(scale_hbm.at[pl.ds(r0, ROWS_PT)], scale_v)
    for g in range(ngroups):
        j = g * NC + c
        pltpu.sync_copy(init_hbm.at[j, pl.ds(r0, ROWS_PT)],
                        acc.at[pl.ds(r0, ROWS_PT)])
        plsc.subcore_barrier()
        pltpu.async_copy(z_hbm.at[j].at[idxs.at[0]], rows0, semg0)

        def body(i, carry):
            b0 = 2 * i
            b1 = 2 * i + 1
            pltpu.make_async_copy(z_hbm.at[j].at[idxs.at[b0]], rows0, semg0).wait()

            @pl.when(i > 0)
            def _():
                pltpu.make_async_copy(rows1, acc.at[idxd.at[0]], sems1).wait()

            pltpu.async_copy(z_hbm.at[j].at[idxs.at[b1]], rows1, semg1)
            pltpu.async_copy(rows0, acc.at[idxd.at[b0]], sems0, add=True)
            pltpu.make_async_copy(z_hbm.at[j].at[idxs.at[b1]], rows1, semg1).wait()

            @pl.when(i < nb // 2 - 1)
            def _():
                pltpu.make_async_copy(rows0, acc.at[idxd.at[0]], sems0).wait()
                pltpu.async_copy(z_hbm.at[j].at[idxs.at[b0 + 2]], rows0, semg0)

            pltpu.async_copy(rows1, acc.at[idxd.at[b1]], sems1, add=True)
            return carry

        lax.fori_loop(0, nb // 2, body, 0)
        pltpu.make_async_copy(rows0, acc.at[idxd.at[0]], sems0).wait()
        pltpu.make_async_copy(rows1, acc.at[idxd.at[0]], sems1).wait()
        plsc.subcore_barrier()
        for m in range(ROWS_PT // FC):
            pltpu.sync_copy(acc.at[pl.ds(r0 + m * FC, FC)],
                            rows0.at[pl.ds(0, FC)])

            def srow(r, carry):
                idxv = jnp.full((16,), m * FC + r, jnp.int32)
                sc = plsc.load_gather(scale_v, [idxv])
                for jj in range(d2 // 16):
                    vec = rows0[r, pl.ds(jj * 16, 16)] * sc
                    if leaky:
                        vec = jnp.where(vec >= 0, vec, 0.01 * vec)
                    rows0[r, pl.ds(jj * 16, 16)] = vec
                return carry

            lax.fori_loop(0, FC, srow, 0)
            pltpu.sync_copy(rows0.at[pl.ds(0, FC)],
                            out_hbm.at[j, pl.ds(r0 + m * FC, FC)])


@functools.cache
def _prop_f_call(d2, ngroups, leaky):
    kb = 1000 if d2 == 32 else 200
    nb = EPT // kb
    mesh = plsc.VectorSubcoreMesh(core_axis_name="c", subcore_axis_name="s")
    return pl.kernel(
        functools.partial(_prop_f_body, ngroups, d2, kb, leaky),
        out_type=jax.ShapeDtypeStruct((ngroups * NC, NNP, d2), jnp.float32),
        mesh=mesh,
        compiler_params=pltpu.CompilerParams(use_tc_tiling_on_sc=False,
                                             needs_layout_passes=False),
        scratch_types=[
            pltpu.VMEM((nb, kb), jnp.int32),
            pltpu.VMEM((nb, kb), jnp.int32),
            pltpu.VMEM((kb, d2), jnp.float32),
            pltpu.VMEM((kb, d2), jnp.float32),
            pltpu.VMEM((ROWS_PT,), jnp.float32),
            pltpu.SemaphoreType.DMA,
            pltpu.SemaphoreType.DMA,
            pltpu.SemaphoreType.DMA,
            pltpu.SemaphoreType.DMA,
            pltpu.VMEM_SHARED((NNP, d2), jnp.float32),
        ],
    )


def _alpha(ek, v):
    logits = jnp.sum(ek * v, axis=1, keepdims=True)
    m = jnp.max(logits)
    e = jnp.exp(logits - m)
    return e / jnp.sum(e)


def _mm_body(pspec, dout, d2, *refs):
    nch = dout // d2
    nparts = len(pspec)
    parts = refs[:nparts]
    ws = refs[nparts:2 * nparts]
    dinv_r, ek_r, v_r, b_r = refs[2 * nparts:2 * nparts + 4]
    z0m_r, z1m_r, s2_r = refs[2 * nparts + 4:]
    acc = None
    for p, w, kind in zip(parts, ws, pspec):
        pv = p[...]
        if kind[0] == '3d':
            pv = pv[0]
        d = jnp.dot(pv, w[...], preferred_element_type=jnp.float32)
        acc = d if acc is None else acc + d
    al = _alpha(ek_r[...], v_r[...])
    dv = dinv_r[...]
    idv = 1.0 / dv
    z0m = acc[:, :dout] * (al[0:1, :] * idv) + idv * b_r[...]
    z1m = acc[:, dout:2 * dout] * (al[1:2, :] * idv)
    s2 = acc[:, 2 * dout:] * (al[2:3, :] * dv)
    for j in range(nch):
        z0m_r[j] = z0m[:, j * d2:(j + 1) * d2]
        z1m_r[j] = z1m[:, j * d2:(j + 1) * d2]
        s2_r[j] = s2[:, j * d2:(j + 1) * d2]


@functools.cache
def _mm_call(pspec, dout, d2):
    nch = dout // d2
    in_specs = []
    for kind in pspec:
        if kind[0] == '2d':
            in_specs.append(pl.BlockSpec((_BM, kind[1]), lambda i: (i, 0)))
        else:
            jj = kind[2]
            in_specs.append(pl.BlockSpec((1, _BM, kind[1]),
                                         lambda i, jj=jj: (jj, i, 0)))
    for kind in pspec:
        in_specs.append(pl.BlockSpec((kind[1], 3 * dout), lambda i: (0, 0)))
    in_specs += [pl.BlockSpec((_BM, 1), lambda i: (i, 0)),
                 pl.BlockSpec((3, 16), lambda i: (0, 0)),
                 pl.BlockSpec((1, 16), lambda i: (0, 0)),
                 pl.BlockSpec((1, dout), lambda i: (0, 0))]
    spec_np = pl.BlockSpec((nch, _BM, d2), lambda i: (0, i, 0))
    out_specs = (spec_np, spec_np, spec_np)
    out_shape = (
        jax.ShapeDtypeStruct((nch, NNP, d2), jnp.float32),
        jax.ShapeDtypeStruct((nch, NNP, d2), jnp.float32),
        jax.ShapeDtypeStruct((nch, NN, d2), jnp.float32),
    )
    return pl.pallas_call(
        functools.partial(_mm_body, pspec, dout, d2),
        grid=(_GRID,),
        in_specs=in_specs,
        out_specs=out_specs,
        out_shape=out_shape,
    )


def _dinv_body(da_r, dinv_r, dvsq_r):
    da = da_r[...]
    deg = da[:NNP, 0:1] + da[NNP:, 0:1]
    dv = lax.rsqrt(jnp.maximum(deg, 1.0))
    dinv_r[...] = dv
    dvsq_r[...] = dv * dv


@functools.cache
def _dinv_call():
    return pl.pallas_call(
        _dinv_body,
        out_shape=(jax.ShapeDtypeStruct((NNP, 1), jnp.float32),
                   jax.ShapeDtypeStruct((NNP, 1), jnp.float32)),
    )


def _layer(parts, wcat, b, ek, v, srcs, dinv, dv1, dv2, dout, d2):
    nch = dout // d2
    ngroups = nch // NC
    kb = 1000 if d2 == 32 else 200
    pspec = []
    ops = []
    wparts = []
    off = 0
    for f in parts:
        if f.ndim == 2:
            dp = f.shape[1]
            pspec.append(('2d', dp))
            ops.append(f)
            wparts.append(wcat[off:off + dp])
            off += dp
        else:
            nchp, _, d2p = f.shape
            for j in range(nchp):
                pspec.append(('3d', d2p, j))
                ops.append(f)
                wparts.append(wcat[off:off + d2p])
                off += d2p
    v2 = v.reshape(1, -1)
    z0m, z1m, s2 = _mm_call(tuple(pspec), dout, d2)(
        *ops, *wparts, dinv, ek, v2, b.reshape(1, -1))
    src2d, dst2d = srcs[kb]
    s1c = _prop_f_call(d2, ngroups, False)(src2d, dst2d, s2, z1m, dv2)
    h = _prop_f_call(d2, ngroups, True)(src2d, dst2d, s1c, z0m, dv1)
    return h


def kernel(x, edge_index, W1, b1, Ek1, v1, W2, b2, Ek2, v2,
           W3, b3, Ek3, v3, W4, b4, Ek4, v4):
    src = edge_index[0]
    dst = edge_index[1]
    srcs = {
        200: (src.reshape(NS * 50, 200), dst.reshape(NS * 50, 200)),
        1000: (src.reshape(NS * 10, 1000), dst.reshape(NS * 10, 1000)),
    }

    ones = jnp.ones((KB_D, 16), jnp.float32)
    zeros16 = jnp.zeros((NNP, 16), jnp.float32)
    deg_acc = _deg_call()(dst.reshape(NS * 50, KB_D), ones, zeros16)
    dinv, dvsq = _dinv_call()(deg_acc)
    dv1 = dinv.reshape(NNP)
    dv2 = dvsq.reshape(NNP)

    W4p = jnp.pad(W4, ((0, 0), (0, 0), (0, 24)))
    b4p = jnp.pad(b4, (0, 24))

    def wcat(W):
        return jnp.concatenate([W[0], W[1], W[2]], axis=1)

    h1 = _layer([x], wcat(W1), b1, Ek1, v1, srcs, dinv, dv1, dv2, 256, 64)
    h2 = _layer([h1, x], wcat(W2), b2, Ek2, v2, srcs, dinv, dv1, dv2, 128, 64)
    h3 = _layer([h2, h1, x], wcat(W3), b3, Ek3, v3, srcs, dinv, dv1, dv2,
                64, 32)
    h4 = _layer([h3, h2, h1, x], wcat(W4p), b4p, Ek4, v4, srcs, dinv, dv1, dv2,
                64, 32)
    return jnp.concatenate([h4[0, :NN], h4[1, :NN]], axis=1)[:, :40]
